# baseline (speedup 1.0000x reference)
import functools
import jax
import jax.numpy as jnp
from jax import lax
from jax.experimental import pallas as pl
from jax.experimental.pallas import tpu as pltpu
from jax.experimental.pallas import tpu_sc as plsc

H = 1024
I = 512
E = 8

NC = 2
NS = 16
NW = NC * NS


def _sc_gather_rows(table, idx, n_rows):
    rows_per_w = n_rows // NW
    mesh = plsc.VectorSubcoreMesh(core_axis_name="c", subcore_axis_name="s")

    @functools.partial(
        pl.kernel, mesh=mesh,
        out_type=jax.ShapeDtypeStruct((n_rows, table.shape[1]), table.dtype),
        scratch_types=[
            pltpu.VMEM((rows_per_w,), jnp.int32),
            pltpu.VMEM((rows_per_w, table.shape[1]), table.dtype),
            pltpu.SemaphoreType.DMA,
        ],
    )
    def gath(table_hbm, idx_hbm, out_hbm, idx_v, rows_v, sem):
        wid = lax.axis_index("s") * NC + lax.axis_index("c")
        base = wid * rows_per_w
        pltpu.sync_copy(idx_hbm.at[pl.ds(base, rows_per_w)], idx_v)
        pltpu.async_copy(table_hbm.at[idx_v], rows_v, sem).wait()
        pltpu.sync_copy(rows_v, out_hbm.at[pl.ds(base, rows_per_w)])

    return gath(table, idx)

_dot = functools.partial(jnp.dot, preferred_element_type=jnp.float32,
                         precision=jax.lax.Precision.DEFAULT)


def _moe_body(x_ref, rw_ref, shg_ref, shu_ref, shd_ref,
              eg_ref, eu_ref, ed_ref, out_ref, mw_ref):
    e = pl.program_id(0)
    x = x_ref[...]
    T = x.shape[0]

    @pl.when(e == 0)
    def _first():
        g = _dot(x, shg_ref[...])
        u = _dot(x, shu_ref[...])
        h = jax.nn.sigmoid(g) * u
        shared = _dot(h, shd_ref[...])
        out_ref[...] = x + shared
        logits = jax.lax.dot_general(
            x, rw_ref[...], (((1,), (1,)), ((), ())),
            preferred_element_type=jnp.float32)
        logits = logits - jnp.max(logits, axis=1, keepdims=True)
        ex = jnp.exp(logits)
        probs = ex / jnp.sum(ex, axis=1, keepdims=True)
        idx8 = jax.lax.broadcasted_iota(jnp.int32, (T, E), 1)
        m1 = jnp.max(probs, axis=1, keepdims=True)
        a1 = jnp.min(jnp.where(probs == m1, idx8, E), axis=1, keepdims=True)
        mask1 = idx8 == a1
        probs2 = jnp.where(mask1, -jnp.inf, probs)
        m2 = jnp.max(probs2, axis=1, keepdims=True)
        a2 = jnp.min(jnp.where(probs2 == m2, idx8, E), axis=1, keepdims=True)
        mask2 = idx8 == a2
        mw_ref[...] = jnp.where(mask1, m1, 0.0) + jnp.where(mask2, m2, 0.0)

    idx8 = jax.lax.broadcasted_iota(jnp.int32, (T, E), 1)
    w_col = jnp.sum(jnp.where(idx8 == e, mw_ref[...], 0.0), axis=1,
                    keepdims=True)
    g = _dot(x, eg_ref[0])
    u = _dot(x, eu_ref[0])
    h = jax.nn.sigmoid(g) * u
    y = _dot(h, ed_ref[0])
    out_ref[...] += y * w_col


def kernel(x, router_w, sh_gate, sh_up, sh_down, exp_gate, exp_up, exp_down):
    Bb, Ss, Hh = x.shape
    flat = x.reshape(-1, Hh)
    T = flat.shape[0]

    flat = _sc_gather_rows(flat, jnp.arange(T, dtype=jnp.int32), T)
    flat = _sc_gather_rows(flat, jnp.arange(T, dtype=jnp.int32), T)

    out = pl.pallas_call(
        _moe_body,
        grid=(E,),
        in_specs=[
            pl.BlockSpec((T, H), lambda e: (0, 0)),
            pl.BlockSpec((E, H), lambda e: (0, 0)),
            pl.BlockSpec((H, I), lambda e: (0, 0)),
            pl.BlockSpec((H, I), lambda e: (0, 0)),
            pl.BlockSpec((I, H), lambda e: (0, 0)),
            pl.BlockSpec((1, H, I), lambda e: (e, 0, 0)),
            pl.BlockSpec((1, H, I), lambda e: (e, 0, 0)),
            pl.BlockSpec((1, I, H), lambda e: (e, 0, 0)),
        ],
        out_specs=pl.BlockSpec((T, H), lambda e: (0, 0)),
        out_shape=jax.ShapeDtypeStruct((T, H), jnp.float32),
        scratch_shapes=[pltpu.VMEM((T, E), jnp.float32)],
    )(flat, router_w, sh_gate, sh_up, sh_down, exp_gate, exp_up, exp_down)

    return out.reshape(Bb, Ss, Hh)


# speedup vs baseline: 1.0627x; 1.0627x over previous
import functools
import jax
import jax.numpy as jnp
from jax.experimental import pallas as pl
from jax.experimental.pallas import tpu as pltpu

H = 1024
I = 512
E = 8

_dot = functools.partial(jnp.dot, preferred_element_type=jnp.float32)


def _moe_body(x_ref, xb_ref, rw_ref, shg_ref, shu_ref, shd_ref,
              eg_ref, eu_ref, ed_ref, out_ref, mw_ref):
    e = pl.program_id(0)
    T = x_ref.shape[0]

    @pl.when(e == 0)
    def _first():
        x = x_ref[...]
        xb = xb_ref[...]
        g = _dot(xb, shg_ref[...])
        u = _dot(xb, shu_ref[...])
        h = (jax.nn.sigmoid(g) * u).astype(jnp.bfloat16)
        shared = _dot(h, shd_ref[...])
        out_ref[...] = x + shared
        logits = jax.lax.dot_general(
            x, rw_ref[...], (((1,), (1,)), ((), ())),
            preferred_element_type=jnp.float32)
        logits = logits - jnp.max(logits, axis=1, keepdims=True)
        ex = jnp.exp(logits)
        probs = ex / jnp.sum(ex, axis=1, keepdims=True)
        idx8 = jax.lax.broadcasted_iota(jnp.int32, (T, E), 1)
        m1 = jnp.max(probs, axis=1, keepdims=True)
        a1 = jnp.min(jnp.where(probs == m1, idx8, E), axis=1, keepdims=True)
        mask1 = idx8 == a1
        probs2 = jnp.where(mask1, -jnp.inf, probs)
        m2 = jnp.max(probs2, axis=1, keepdims=True)
        a2 = jnp.min(jnp.where(probs2 == m2, idx8, E), axis=1, keepdims=True)
        mask2 = idx8 == a2
        mw_ref[...] = jnp.where(mask1, m1, 0.0) + jnp.where(mask2, m2, 0.0)

    idx8 = jax.lax.broadcasted_iota(jnp.int32, (T, E), 1)
    w_col = jnp.sum(jnp.where(idx8 == e, mw_ref[...], 0.0), axis=1,
                    keepdims=True)
    xb = xb_ref[...]
    g = _dot(xb, eg_ref[0])
    u = _dot(xb, eu_ref[0])
    h = (jax.nn.sigmoid(g) * u).astype(jnp.bfloat16)
    y = _dot(h, ed_ref[0])
    out_ref[...] += y * w_col


def kernel(x, router_w, sh_gate, sh_up, sh_down, exp_gate, exp_up, exp_down):
    Bb, Ss, Hh = x.shape
    flat = x.reshape(-1, Hh)
    T = flat.shape[0]
    b16 = jnp.bfloat16
    flatb = flat.astype(b16)

    out = pl.pallas_call(
        _moe_body,
        grid=(E,),
        in_specs=[
            pl.BlockSpec((T, H), lambda e: (0, 0)),
            pl.BlockSpec((T, H), lambda e: (0, 0)),
            pl.BlockSpec((E, H), lambda e: (0, 0)),
            pl.BlockSpec((H, I), lambda e: (0, 0)),
            pl.BlockSpec((H, I), lambda e: (0, 0)),
            pl.BlockSpec((I, H), lambda e: (0, 0)),
            pl.BlockSpec((1, H, I), lambda e: (e, 0, 0)),
            pl.BlockSpec((1, H, I), lambda e: (e, 0, 0)),
            pl.BlockSpec((1, I, H), lambda e: (e, 0, 0)),
        ],
        out_specs=pl.BlockSpec((T, H), lambda e: (0, 0)),
        out_shape=jax.ShapeDtypeStruct((T, H), jnp.float32),
        scratch_shapes=[pltpu.VMEM((T, E), jnp.float32)],
    )(flat, flatb, router_w, sh_gate.astype(b16), sh_up.astype(b16),
      sh_down.astype(b16), exp_gate.astype(b16), exp_up.astype(b16),
      exp_down.astype(b16))

    return out.reshape(Bb, Ss, Hh)


# speedup vs baseline: 1.4123x; 1.3290x over previous
import functools
import jax
import jax.numpy as jnp
from jax.experimental import pallas as pl
from jax.experimental.pallas import tpu as pltpu

H = 1024
I = 512
E = 8

_dot = functools.partial(jnp.dot, preferred_element_type=jnp.float32,
                         precision=jax.lax.Precision.DEFAULT)


def _moe_body(x_ref, rw_ref, shg_ref, shu_ref, shd_ref,
              eg_ref, eu_ref, ed_ref, out_ref, mw_ref, g_ref, u_ref):
    s = pl.program_id(0)
    x = x_ref[...]
    T = x.shape[0]

    @pl.when(s == 0)
    def _first():
        g = _dot(x, shg_ref[...])
        u = _dot(x, shu_ref[...])
        h = jax.nn.sigmoid(g) * u
        shared = _dot(h, shd_ref[...])
        out_ref[...] = x + shared
        logits = jax.lax.dot_general(
            x, rw_ref[...], (((1,), (1,)), ((), ())),
            preferred_element_type=jnp.float32)
        logits = logits - jnp.max(logits, axis=1, keepdims=True)
        ex = jnp.exp(logits)
        probs = ex / jnp.sum(ex, axis=1, keepdims=True)
        idx8 = jax.lax.broadcasted_iota(jnp.int32, (T, E), 1)
        m1 = jnp.max(probs, axis=1, keepdims=True)
        a1 = jnp.min(jnp.where(probs == m1, idx8, E), axis=1, keepdims=True)
        mask1 = idx8 == a1
        probs2 = jnp.where(mask1, -jnp.inf, probs)
        m2 = jnp.max(probs2, axis=1, keepdims=True)
        a2 = jnp.min(jnp.where(probs2 == m2, idx8, E), axis=1, keepdims=True)
        mask2 = idx8 == a2
        mw_ref[...] = jnp.where(mask1, m1, 0.0) + jnp.where(mask2, m2, 0.0)

    @pl.when(s > 0)
    def _finish():
        e = s - 1
        idx8 = jax.lax.broadcasted_iota(jnp.int32, (T, E), 1)
        w_col = jnp.sum(jnp.where(idx8 == e, mw_ref[...], 0.0), axis=1,
                        keepdims=True)
        h = jax.nn.sigmoid(g_ref[...]) * u_ref[...]
        y = _dot(h, ed_ref[0])
        out_ref[...] += y * w_col

    @pl.when(s < E)
    def _start():
        g_ref[...] = _dot(x, eg_ref[0])
        u_ref[...] = _dot(x, eu_ref[0])


def kernel(x, router_w, sh_gate, sh_up, sh_down, exp_gate, exp_up, exp_down):
    Bb, Ss, Hh = x.shape
    flat = x.reshape(-1, Hh)
    T = flat.shape[0]

    out = pl.pallas_call(
        _moe_body,
        grid=(E + 1,),
        in_specs=[
            pl.BlockSpec((T, H), lambda s: (0, 0)),
            pl.BlockSpec((E, H), lambda s: (0, 0)),
            pl.BlockSpec((H, I), lambda s: (0, 0)),
            pl.BlockSpec((H, I), lambda s: (0, 0)),
            pl.BlockSpec((I, H), lambda s: (0, 0)),
            pl.BlockSpec((1, H, I), lambda s: (jnp.minimum(s, E - 1), 0, 0)),
            pl.BlockSpec((1, H, I), lambda s: (jnp.minimum(s, E - 1), 0, 0)),
            pl.BlockSpec((1, I, H),
                         lambda s: (jnp.maximum(s - 1, 0), 0, 0)),
        ],
        out_specs=pl.BlockSpec((T, H), lambda s: (0, 0)),
        out_shape=jax.ShapeDtypeStruct((T, H), jnp.float32),
        scratch_shapes=[
            pltpu.VMEM((T, E), jnp.float32),
            pltpu.VMEM((T, I), jnp.float32),
            pltpu.VMEM((T, I), jnp.float32),
        ],
    )(flat, router_w, sh_gate, sh_up, sh_down, exp_gate, exp_up, exp_down)

    return out.reshape(Bb, Ss, Hh)


# speedup vs baseline: 1.4273x; 1.0106x over previous
import functools
import jax
import jax.numpy as jnp
from jax.experimental import pallas as pl
from jax.experimental.pallas import tpu as pltpu

H = 1024
I = 512
E = 8

_dot = functools.partial(jnp.dot, preferred_element_type=jnp.float32,
                         precision=jax.lax.Precision.DEFAULT)


def _moe_body(x_ref, rw_ref, shg_ref, shu_ref, shd_ref,
              eg_ref, eu_ref, ed_ref, out_ref, mw_ref):
    e = pl.program_id(0)
    x = x_ref[...]
    T = x.shape[0]

    @pl.when(e == 0)
    def _first():
        g = _dot(x, shg_ref[...])
        u = _dot(x, shu_ref[...])
        h = jax.nn.sigmoid(g) * u
        shared = _dot(h, shd_ref[...])
        out_ref[...] = x + shared
        logits = jax.lax.dot_general(
            x, rw_ref[...], (((1,), (1,)), ((), ())),
            preferred_element_type=jnp.float32)
        logits = logits - jnp.max(logits, axis=1, keepdims=True)
        ex = jnp.exp(logits)
        probs = ex / jnp.sum(ex, axis=1, keepdims=True)
        idx8 = jax.lax.broadcasted_iota(jnp.int32, (T, E), 1)
        m1 = jnp.max(probs, axis=1, keepdims=True)
        a1 = jnp.min(jnp.where(probs == m1, idx8, E), axis=1, keepdims=True)
        mask1 = idx8 == a1
        probs2 = jnp.where(mask1, -jnp.inf, probs)
        m2 = jnp.max(probs2, axis=1, keepdims=True)
        a2 = jnp.min(jnp.where(probs2 == m2, idx8, E), axis=1, keepdims=True)
        mask2 = idx8 == a2
        mw_ref[...] = jnp.where(mask1, m1, 0.0) + jnp.where(mask2, m2, 0.0)

    idx8 = jax.lax.broadcasted_iota(jnp.int32, (T, E), 1)
    w_col = jnp.sum(jnp.where(idx8 == e, mw_ref[...], 0.0), axis=1,
                    keepdims=True)
    g = _dot(x, eg_ref[0])
    u = _dot(x, eu_ref[0])
    h = jax.nn.sigmoid(g) * u
    y = _dot(h, ed_ref[0])
    out_ref[...] += y * w_col


def kernel(x, router_w, sh_gate, sh_up, sh_down, exp_gate, exp_up, exp_down):
    Bb, Ss, Hh = x.shape
    flat = x.reshape(-1, Hh)
    T = flat.shape[0]

    out = pl.pallas_call(
        _moe_body,
        grid=(E,),
        in_specs=[
            pl.BlockSpec((T, H), lambda e: (0, 0)),
            pl.BlockSpec((E, H), lambda e: (0, 0)),
            pl.BlockSpec((H, I), lambda e: (0, 0)),
            pl.BlockSpec((H, I), lambda e: (0, 0)),
            pl.BlockSpec((I, H), lambda e: (0, 0)),
            pl.BlockSpec((1, H, I), lambda e: (e, 0, 0)),
            pl.BlockSpec((1, H, I), lambda e: (e, 0, 0)),
            pl.BlockSpec((1, I, H), lambda e: (e, 0, 0)),
        ],
        out_specs=pl.BlockSpec((T, H), lambda e: (0, 0)),
        out_shape=jax.ShapeDtypeStruct((T, H), jnp.float32),
        scratch_shapes=[pltpu.VMEM((T, E), jnp.float32)],
    )(flat, router_w, sh_gate, sh_up, sh_down, exp_gate, exp_up, exp_down)

    return out.reshape(Bb, Ss, Hh)
